# baseline (speedup 1.0000x reference)
import jax
import jax.numpy as jnp
from jax.experimental import pallas as pl
from jax.experimental.pallas import tpu as pltpu

D_MODEL = 1024
D_SPACE = 64
S = 2048
N_OUT = 20480
TN = 1024
NUM_J = N_OUT // TN
XK = D_MODEL // 4

_STEPS = (
    (12, 16, 7, 1), (13, 17, 7, 1), (14, 18, 7, 1), (15, 19, 7, 1),
    (4, 10, 4, 1), (4, 12, 5, 0), (5, 11, 4, 1), (5, 13, 5, 0),
    (6, 14, 6, 1), (7, 15, 6, 1),
    (0, 0, 0, 1), (0, 2, 1, 0), (1, 1, 0, 1), (1, 3, 1, 0),
    (2, 4, 2, 1), (3, 5, 2, 1),
    (8, 6, 3, 1), (9, 7, 3, 1), (10, 8, 3, 1), (11, 9, 3, 1),
)
_NTAB = tuple(t[0] for t in _STEPS)
_OTAB = tuple(t[1] for t in _STEPS)
_HTAB = tuple(t[2] for t in _STEPS)
_FTAB = tuple(t[3] for t in _STEPS)


def _group_mean_mat(n):
    r = jax.lax.broadcasted_iota(jnp.int32, (n, n), 0) // D_SPACE
    c = jax.lax.broadcasted_iota(jnp.int32, (n, n), 1) // D_SPACE
    return jnp.where(r == c, 1.0 / D_SPACE, 0.0).astype(jnp.float32)


def _ln_heads(scr, k0, t, g, b):
    n = t.shape[-1]
    gm = _group_mean_mat(n)
    m = jnp.dot(t, gm, preferred_element_type=jnp.float32)
    ms = jnp.dot(t * t, gm, preferred_element_type=jnp.float32)
    v = ms - m * m
    h = ((t - m) * jax.lax.rsqrt(v + 1e-5) * g + b).astype(jnp.bfloat16)
    for k in range(n // D_SPACE):
        scr[k0 + k] = h[:, k * D_SPACE:(k + 1) * D_SPACE]


def _body(tab_ref, x_ref, ca_ref, ck_ref, ne_ref,
          wf_ref, wk_ref, wrq_ref, wrk_ref, wrv_ref, wkn_ref,
          bf_ref, bk_ref, brq_ref, brk_ref, brv_ref, bkn_ref,
          g0_ref, b0_ref, g1_ref, b1_ref, g2_ref, b2_ref, g3_ref, b3_ref,
          g4_ref, b4_ref, g5_ref, b5_ref, g6_ref, b6_ref, g7_ref, b7_ref,
          out_ref, h_scr, px_scr, en_scr):
    s = pl.program_id(0)

    @pl.when(s == 0)
    def _know_prologue():
        pk = jnp.dot(ck_ref[...].astype(jnp.bfloat16),
                     wkn_ref[...].astype(jnp.bfloat16),
                     preferred_element_type=jnp.float32) + bkn_ref[...]
        _ln_heads(h_scr, 7, pk, g7_ref[...], b7_ref[...])

    @pl.when(s == 1)
    def _attn_prologue():
        wr = jnp.concatenate(
            [wrq_ref[...], wrk_ref[...], wrv_ref[...]],
            axis=1).astype(jnp.bfloat16)
        br = jnp.concatenate([brq_ref[...], brk_ref[...], brv_ref[...]],
                             axis=1)
        pr = jnp.dot(ca_ref[...].astype(jnp.bfloat16), wr,
                     preferred_element_type=jnp.float32) + br
        g = jnp.concatenate([g4_ref[...], g5_ref[...], g6_ref[...]], axis=1)
        b = jnp.concatenate([b4_ref[...], b5_ref[...], b6_ref[...]], axis=1)
        _ln_heads(h_scr, 4, pr, g, b)

    for q in range(4):
        @pl.when(s == 2 + q)
        def _x_prologue_q(q=q):
            wq = jnp.concatenate(
                [wf_ref[q * XK:(q + 1) * XK, :],
                 wk_ref[q * XK:(q + 1) * XK, :]],
                axis=1).astype(jnp.bfloat16)
            part = jnp.dot(x_ref[...].astype(jnp.bfloat16), wq,
                           preferred_element_type=jnp.float32)
            if q == 0:
                px_scr[...] = part
            else:
                px_scr[...] += part

    @pl.when(s == 5)
    def _x_heads():
        bx = jnp.concatenate([bf_ref[...], bk_ref[...]], axis=1)
        px = px_scr[...] + bx
        g = jnp.concatenate([g0_ref[...], g1_ref[...], g2_ref[...],
                             g3_ref[...]], axis=1)
        b = jnp.concatenate([b0_ref[...], b1_ref[...], b2_ref[...],
                             b3_ref[...]], axis=1)
        _ln_heads(h_scr, 0, px, g, b)

    h = h_scr[tab_ref[2, s]]
    dims = (((1,), (1,)), ((), ()))

    @pl.when(tab_ref[3, s] == 1)
    def _fresh_block():
        e = ne_ref[...]
        s2 = jnp.dot(e * e, _group_mean_mat(D_SPACE) * D_SPACE,
                     preferred_element_type=jnp.float32)
        inv = 1.0 / jnp.maximum(jnp.sqrt(s2), 1e-12)
        en = (e * inv).astype(jnp.bfloat16)
        en_scr[...] = en
        out_ref[...] = jax.lax.dot_general(
            h, en, dims, preferred_element_type=jnp.float32)

    @pl.when(tab_ref[3, s] == 0)
    def _reused_block():
        out_ref[...] = jax.lax.dot_general(
            h, en_scr[...], dims, preferred_element_type=jnp.float32)


def kernel(x, ctx_attn, ctx_know, neuron_emb, W_feat, b_feat, W_know, b_know,
           W_rQ, b_rQ, W_rK, b_rK, W_rV, b_rV, W_rKn, b_rKn,
           g_fqkQ, beta_fqkQ, g_fqkK, beta_fqkK, g_fv, beta_fv,
           g_fkn, beta_fkn, g_rQ, beta_rQ, g_rK, beta_rK,
           g_rV, beta_rV, g_rKn, beta_rKn):
    B = x.shape[0]
    x2 = x.reshape(B * S, D_MODEL)
    ca = ctx_attn.reshape(B * S, -1)
    ck = ctx_know.reshape(B * S, -1)
    row = lambda a: a[None, :]

    tab = jnp.asarray([_NTAB, _OTAB, _HTAB, _FTAB],
                      dtype=jnp.int32)
    full = lambda a: pl.BlockSpec(a.shape, lambda s, t: (0,) * a.ndim)

    small = [W_feat, W_know, W_rQ, W_rK, W_rV, W_rKn,
             row(b_feat), row(b_know), row(b_rQ), row(b_rK), row(b_rV),
             row(b_rKn),
             row(g_fqkQ), row(beta_fqkQ), row(g_fqkK), row(beta_fqkK),
             row(g_fv), row(beta_fv), row(g_fkn), row(beta_fkn),
             row(g_rQ), row(beta_rQ), row(g_rK), row(beta_rK),
             row(g_rV), row(beta_rV), row(g_rKn), row(beta_rKn)]

    grid_spec = pltpu.PrefetchScalarGridSpec(
        num_scalar_prefetch=1,
        grid=(NUM_J,),
        in_specs=[
            pl.BlockSpec((B * S, XK),
                         lambda s, t: (0, jnp.clip(s - 2, 0, 3))),
            full(ca), full(ck),
            pl.BlockSpec((TN, D_SPACE), lambda s, t: (t[0, s], 0)),
        ] + [full(a) for a in small],
        out_specs=pl.BlockSpec((B * S, TN), lambda s, t: (0, t[1, s])),
        scratch_shapes=[pltpu.VMEM((8, B * S, D_SPACE), jnp.bfloat16),
                        pltpu.VMEM((B * S, 256), jnp.float32),
                        pltpu.VMEM((TN, D_SPACE), jnp.bfloat16)],
    )

    out = pl.pallas_call(
        _body,
        grid_spec=grid_spec,
        out_shape=jax.ShapeDtypeStruct((B * S, N_OUT), jnp.float32),
    )(tab, x2, ca, ck, neuron_emb, *small)

    return out.reshape(B, S, N_OUT)


# speedup vs baseline: 1.0027x; 1.0027x over previous
import jax
import jax.numpy as jnp
from jax.experimental import pallas as pl
from jax.experimental.pallas import tpu as pltpu

D_MODEL = 1024
D_SPACE = 64
S = 2048
N_OUT = 20480
TN = 1024
NUM_J = N_OUT // TN
XK = D_MODEL // 4

_STEPS = (
    (12, 16, 7, 1), (13, 17, 7, 1), (14, 18, 7, 1), (15, 19, 7, 1),
    (4, 10, 4, 1), (4, 12, 5, 0), (5, 11, 4, 1), (5, 13, 5, 0),
    (6, 14, 6, 1), (7, 15, 6, 1),
    (0, 0, 0, 1), (0, 2, 1, 0), (1, 1, 0, 1), (1, 3, 1, 0),
    (2, 4, 2, 1), (3, 5, 2, 1),
    (8, 6, 3, 1), (9, 7, 3, 1), (10, 8, 3, 1), (11, 9, 3, 1),
)
_NTAB = tuple(t[0] for t in _STEPS)
_OTAB = tuple(t[1] for t in _STEPS)
_HTAB = tuple(t[2] for t in _STEPS)
_FTAB = tuple(t[3] for t in _STEPS)


def _group_mean_mat(n):
    r = jax.lax.broadcasted_iota(jnp.int32, (n, n), 0) // D_SPACE
    c = jax.lax.broadcasted_iota(jnp.int32, (n, n), 1) // D_SPACE
    return jnp.where(r == c, 1.0 / D_SPACE, 0.0).astype(jnp.float32)


def _ln_heads(scr, k0, t, g, b):
    n = t.shape[-1]
    gm = _group_mean_mat(n)
    m = jnp.dot(t, gm, preferred_element_type=jnp.float32)
    ms = jnp.dot(t * t, gm, preferred_element_type=jnp.float32)
    v = ms - m * m
    h = ((t - m) * jax.lax.rsqrt(v + 1e-5) * g + b).astype(jnp.bfloat16)
    for k in range(n // D_SPACE):
        scr[k0 + k] = h[:, k * D_SPACE:(k + 1) * D_SPACE]


def _body(tab_ref, x_ref, ca_ref, ck_ref, ne_ref,
          wf_ref, wk_ref, wrq_ref, wrk_ref, wrv_ref, wkn_ref,
          bf_ref, bk_ref, brq_ref, brk_ref, brv_ref, bkn_ref,
          g0_ref, b0_ref, g1_ref, b1_ref, g2_ref, b2_ref, g3_ref, b3_ref,
          g4_ref, b4_ref, g5_ref, b5_ref, g6_ref, b6_ref, g7_ref, b7_ref,
          out_ref, h_scr, px_scr, en_scr):
    s = pl.program_id(0)

    @pl.when(s == 0)
    def _know_prologue():
        pk = jnp.dot(ck_ref[...].astype(jnp.bfloat16),
                     wkn_ref[...].astype(jnp.bfloat16),
                     preferred_element_type=jnp.float32) + bkn_ref[...]
        _ln_heads(h_scr, 7, pk, g7_ref[...], b7_ref[...])

    @pl.when(s == 1)
    def _attn_prologue():
        wr = jnp.concatenate(
            [wrq_ref[...], wrk_ref[...], wrv_ref[...]],
            axis=1).astype(jnp.bfloat16)
        br = jnp.concatenate([brq_ref[...], brk_ref[...], brv_ref[...]],
                             axis=1)
        pr = jnp.dot(ca_ref[...].astype(jnp.bfloat16), wr,
                     preferred_element_type=jnp.float32) + br
        g = jnp.concatenate([g4_ref[...], g5_ref[...], g6_ref[...]], axis=1)
        b = jnp.concatenate([b4_ref[...], b5_ref[...], b6_ref[...]], axis=1)
        _ln_heads(h_scr, 4, pr, g, b)

    for q in range(4):
        @pl.when(s == 2 + q)
        def _x_prologue_q(q=q):
            wq = jnp.concatenate(
                [wf_ref[q * XK:(q + 1) * XK, :],
                 wk_ref[q * XK:(q + 1) * XK, :]],
                axis=1).astype(jnp.bfloat16)
            part = jnp.dot(x_ref[...].astype(jnp.bfloat16), wq,
                           preferred_element_type=jnp.float32)
            if q == 0:
                px_scr[...] = part
            else:
                px_scr[...] += part

    @pl.when(s == 5)
    def _x_heads():
        bx = jnp.concatenate([bf_ref[...], bk_ref[...]], axis=1)
        px = px_scr[...] + bx
        g = jnp.concatenate([g0_ref[...], g1_ref[...], g2_ref[...],
                             g3_ref[...]], axis=1)
        b = jnp.concatenate([b0_ref[...], b1_ref[...], b2_ref[...],
                             b3_ref[...]], axis=1)
        _ln_heads(h_scr, 0, px, g, b)

    @pl.when(tab_ref[3, s] == 1)
    def _normalize_block():
        e = ne_ref[...]
        s2 = jnp.dot(e * e, _group_mean_mat(D_SPACE) * D_SPACE,
                     preferred_element_type=jnp.float32)
        inv = 1.0 / jnp.maximum(jnp.sqrt(s2), 1e-12)
        en_scr[...] = (e * inv).astype(jnp.bfloat16)

    h = h_scr[tab_ref[2, s]]
    out_ref[...] = jax.lax.dot_general(
        h, en_scr[...], (((1,), (1,)), ((), ())),
        preferred_element_type=jnp.float32)


def kernel(x, ctx_attn, ctx_know, neuron_emb, W_feat, b_feat, W_know, b_know,
           W_rQ, b_rQ, W_rK, b_rK, W_rV, b_rV, W_rKn, b_rKn,
           g_fqkQ, beta_fqkQ, g_fqkK, beta_fqkK, g_fv, beta_fv,
           g_fkn, beta_fkn, g_rQ, beta_rQ, g_rK, beta_rK,
           g_rV, beta_rV, g_rKn, beta_rKn):
    B = x.shape[0]
    x2 = x.reshape(B * S, D_MODEL)
    ca = ctx_attn.reshape(B * S, -1)
    ck = ctx_know.reshape(B * S, -1)
    row = lambda a: a[None, :]

    tab = jnp.asarray([_NTAB, _OTAB, _HTAB, _FTAB],
                      dtype=jnp.int32)
    full = lambda a: pl.BlockSpec(a.shape, lambda s, t: (0,) * a.ndim)

    small = [W_feat, W_know, W_rQ, W_rK, W_rV, W_rKn,
             row(b_feat), row(b_know), row(b_rQ), row(b_rK), row(b_rV),
             row(b_rKn),
             row(g_fqkQ), row(beta_fqkQ), row(g_fqkK), row(beta_fqkK),
             row(g_fv), row(beta_fv), row(g_fkn), row(beta_fkn),
             row(g_rQ), row(beta_rQ), row(g_rK), row(beta_rK),
             row(g_rV), row(beta_rV), row(g_rKn), row(beta_rKn)]

    grid_spec = pltpu.PrefetchScalarGridSpec(
        num_scalar_prefetch=1,
        grid=(NUM_J,),
        in_specs=[
            pl.BlockSpec((B * S, XK),
                         lambda s, t: (0, jnp.clip(s - 2, 0, 3))),
            full(ca), full(ck),
            pl.BlockSpec((TN, D_SPACE), lambda s, t: (t[0, s], 0)),
        ] + [full(a) for a in small],
        out_specs=pl.BlockSpec((B * S, TN), lambda s, t: (0, t[1, s])),
        scratch_shapes=[pltpu.VMEM((8, B * S, D_SPACE), jnp.bfloat16),
                        pltpu.VMEM((B * S, 256), jnp.float32),
                        pltpu.VMEM((TN, D_SPACE), jnp.bfloat16)],
    )

    out = pl.pallas_call(
        _body,
        grid_spec=grid_spec,
        out_shape=jax.ShapeDtypeStruct((B * S, N_OUT), jnp.float32),
    )(tab, x2, ca, ck, neuron_emb, *small)

    return out.reshape(B, S, N_OUT)
